# baseline (speedup 1.0000x reference)
import functools

import jax
import jax.numpy as jnp
from jax import lax
from jax.experimental import pallas as pl
from jax.experimental.pallas import tpu as pltpu
from jax.experimental.pallas import tpu_sc as plsc

_D = 4096
_B = 16384
_K = 8
_NSUB = 2 * _K
_NC = 2
_NS = 16
_NW = _NC * _NS
_CPT = _D // _NW
_R = 128

_MIN_BW = 1e-3
_MIN_D = 1e-3
_MIN_L = 0.025
_BOUND = 3.0

_BSC = 9472
_TCR = 256
_NCH = _BSC // _R


def _prep_body(uw_ref, uh_ref, ud_ref, ul_ref, tab_ref):
    w8 = jax.nn.softmax(uw_ref[:], axis=0)
    h8 = jax.nn.softmax(uh_ref[:], axis=0)
    dv = _MIN_D + jax.nn.softplus(ud_ref[:])
    lam8 = (1.0 - 2.0 * _MIN_L) * jax.nn.sigmoid(ul_ref[:]) + _MIN_L

    w8 = _MIN_BW + (1.0 - _MIN_BW * _K) * w8
    h8 = _MIN_BW + (1.0 - _MIN_BW * _K) * h8

    one = jnp.ones((1, _D), jnp.float32)

    cw = [jnp.full((1, _D), -_BOUND, jnp.float32)]
    acc = jnp.zeros((1, _D), jnp.float32)
    for k in range(_K - 1):
        acc = acc + w8[k:k + 1, :]
        cw.append(2.0 * _BOUND * acc - _BOUND)
    cw.append(jnp.full((1, _D), _BOUND, jnp.float32))

    ch = [jnp.full((1, _D), -_BOUND, jnp.float32)]
    acc = jnp.zeros((1, _D), jnp.float32)
    for k in range(_K - 1):
        acc = acc + h8[k:k + 1, :]
        ch.append(2.0 * _BOUND * acc - _BOUND)
    ch.append(jnp.full((1, _D), _BOUND, jnp.float32))

    dpad = [one] + [dv[k:k + 1, :] for k in range(_K - 1)] + [one]

    for k in range(_K):
        wk = cw[k + 1] - cw[k]
        hk = ch[k + 1] - ch[k]
        lam = lam8[k:k + 1, :]
        dk = dpad[k]
        dk1 = dpad[k + 1]
        wb = jnp.sqrt(dk / dk1)
        delta = hk / wk
        wc = (lam * dk + (1.0 - lam) * wb * dk1) / delta
        ya = ch[k]
        yb = hk + ch[k]
        yc = ((1.0 - lam) * ya + lam * wb * yb) / ((1.0 - lam) + lam * wb)

        lamw = lam * wk
        tab_ref[pl.ds(2 * k, 1), :] = cw[k]
        tab_ref[pl.ds(_NSUB + 2 * k, 1), :] = ya * lamw
        tab_ref[pl.ds(2 * _NSUB + 2 * k, 1), :] = wc * yc - ya
        tab_ref[pl.ds(3 * _NSUB + 2 * k, 1), :] = lamw
        tab_ref[pl.ds(4 * _NSUB + 2 * k, 1), :] = wc - 1.0
        wcyc = wc * yc
        wbyb = wb * yb
        om = wk - lamw
        tab_ref[pl.ds(2 * k + 1, 1), :] = cw[k] + lamw
        tab_ref[pl.ds(_NSUB + 2 * k + 1, 1), :] = wcyc * om
        tab_ref[pl.ds(2 * _NSUB + 2 * k + 1, 1), :] = wbyb - wcyc
        tab_ref[pl.ds(3 * _NSUB + 2 * k + 1, 1), :] = wc * om
        tab_ref[pl.ds(4 * _NSUB + 2 * k + 1, 1), :] = wb - wc


def _build_tables(uw, uh, ud, ul):
    return pl.pallas_call(
        _prep_body,
        out_shape=jax.ShapeDtypeStruct((5 * _NSUB, _D), jnp.float32),
    )(uw.T, uh.T, ud.T, ul.T)


def _sc_body(x_hbm, tab_hbm, out_hbm, tab_v, in0, in1, out0, out1,
             is0, is1, os0, os1):
    wid = lax.axis_index("s") * _NC + lax.axis_index("c")
    col0 = wid * _CPT

    pltpu.sync_copy(tab_hbm.at[wid], tab_v)

    def in_dma(g, buf, sem):
        pltpu.async_copy(
            x_hbm.at[pl.ds(g * _R, _R), pl.ds(col0, _CPT)], buf, sem)

    def out_dma(g, buf, sem):
        pltpu.async_copy(
            buf, out_hbm.at[pl.ds(g * _R, _R), pl.ds(col0, _CPT)], sem)

    def wait_in(buf, sem):
        pltpu.make_async_copy(
            x_hbm.at[pl.ds(0, _R), pl.ds(col0, _CPT)], buf, sem).wait()

    def wait_out(buf, sem):
        pltpu.make_async_copy(
            buf, out_hbm.at[pl.ds(0, _R), pl.ds(col0, _CPT)], sem).wait()

    in_dma(0, in0, is0)
    in_dma(1, in1, is1)

    lanes = jnp.arange(16, dtype=jnp.int32)
    half = jnp.full((16,), 8 * _CPT, jnp.int32)
    zeros = jnp.zeros((16,), jnp.int32)

    neg3 = jnp.full((16,), -_BOUND, jnp.float32)

    tab_p = tab_v.at[pl.ds(_NSUB * _CPT, _NSUB * _CPT)]
    tab_q = tab_v.at[pl.ds(2 * _NSUB * _CPT, _NSUB * _CPT)]
    tab_r = tab_v.at[pl.ds(3 * _NSUB * _CPT, _NSUB * _CPT)]
    tab_s = tab_v.at[pl.ds(4 * _NSUB * _CPT, _NSUB * _CPT)]

    def compute_chunk(ibuf, obuf):
        for g in range(_CPT // 16):
            colv = lanes + (g * 16)
            colv_hi = colv + 8 * _CPT
            b8 = tab_v[pl.ds(8 * _CPT + g * 16, 16)]

            @plsc.parallel_loop(0, _R, unroll=4)
            def row(i, g=g, colv=colv, colv_hi=colv_hi, b8=b8):
                x = ibuf[i, pl.ds(g * 16, 16)]
                m = x >= b8
                jc = jnp.where(m, colv_hi, colv)
                blo = jnp.where(m, b8, neg3)
                for step in (4 * _CPT, 2 * _CPT, _CPT):
                    probe = jc + step
                    bv = plsc.load_gather(tab_v, [probe])
                    m = x >= bv
                    jc = jnp.where(m, probe, jc)
                    blo = jnp.where(m, bv, blo)
                pv = plsc.load_gather(tab_p, [jc])
                qv = plsc.load_gather(tab_q, [jc])
                rv = plsc.load_gather(tab_r, [jc])
                sv = plsc.load_gather(tab_s, [jc])
                t = x - blo
                y = (pv + qv * t) / (rv + sv * t)
                ok = jnp.abs(x) <= _BOUND
                obuf[i, pl.ds(g * 16, 16)] = jnp.where(ok, y, x)

    def pair(pi, carry):
        for g_off, ibuf, obuf, isem, osem in (
                (0, in0, out0, is0, os0), (1, in1, out1, is1, os1)):
            g = pi * 2 + g_off
            wait_in(ibuf, isem)

            @pl.when(pi >= 1)
            def _():
                wait_out(obuf, osem)

            compute_chunk(ibuf, obuf)
            out_dma(g, obuf, osem)

            @pl.when(g + 2 < _NCH)
            def _():
                in_dma(g + 2, ibuf, isem)
        return carry

    lax.fori_loop(0, _NCH // 2, pair, 0)
    wait_out(out0, os0)
    wait_out(out1, os1)


@functools.cache
def _make_sc_spline():
    return functools.partial(
        pl.kernel,
        out_type=jax.ShapeDtypeStruct((_B, _D), jnp.float32),
        mesh=plsc.VectorSubcoreMesh(core_axis_name="c", subcore_axis_name="s"),
        compiler_params=pltpu.CompilerParams(needs_layout_passes=False),
        scratch_types=[
            pltpu.VMEM((5 * _NSUB * _CPT,), jnp.float32),
            pltpu.VMEM((_R, _CPT), jnp.float32),
            pltpu.VMEM((_R, _CPT), jnp.float32),
            pltpu.VMEM((_R, _CPT), jnp.float32),
            pltpu.VMEM((_R, _CPT), jnp.float32),
            pltpu.SemaphoreType.DMA,
            pltpu.SemaphoreType.DMA,
            pltpu.SemaphoreType.DMA,
            pltpu.SemaphoreType.DMA,
        ],
    )(_sc_body)


def _tc_spline_body(x_ref, tab_ref, o_ref):
    x = x_ref[:]
    ind = x >= tab_ref[pl.ds(1, 1), :]
    bs = jnp.where(ind, tab_ref[pl.ds(1, 1), :], tab_ref[pl.ds(0, 1), :])
    ps = jnp.where(ind, tab_ref[pl.ds(_NSUB + 1, 1), :],
                   tab_ref[pl.ds(_NSUB, 1), :])
    qs = jnp.where(ind, tab_ref[pl.ds(2 * _NSUB + 1, 1), :],
                   tab_ref[pl.ds(2 * _NSUB, 1), :])
    rs = jnp.where(ind, tab_ref[pl.ds(3 * _NSUB + 1, 1), :],
                   tab_ref[pl.ds(3 * _NSUB, 1), :])
    ss = jnp.where(ind, tab_ref[pl.ds(4 * _NSUB + 1, 1), :],
                   tab_ref[pl.ds(4 * _NSUB, 1), :])
    for m in range(2, _NSUB):
        ind = x >= tab_ref[pl.ds(m, 1), :]
        bs = jnp.where(ind, tab_ref[pl.ds(m, 1), :], bs)
        ps = jnp.where(ind, tab_ref[pl.ds(_NSUB + m, 1), :], ps)
        qs = jnp.where(ind, tab_ref[pl.ds(2 * _NSUB + m, 1), :], qs)
        rs = jnp.where(ind, tab_ref[pl.ds(3 * _NSUB + m, 1), :], rs)
        ss = jnp.where(ind, tab_ref[pl.ds(4 * _NSUB + m, 1), :], ss)
    t = x - bs
    y = (ps + qs * t) / (rs + ss * t)
    ok = jnp.abs(x) <= _BOUND
    o_ref[:] = jnp.where(ok, y, x)


def _tc_spline(x, tab):
    n_blocks = (_B - _BSC) // _TCR
    return pl.pallas_call(
        _tc_spline_body,
        grid=(n_blocks,),
        in_specs=[
            pl.BlockSpec((_TCR, _D), lambda i: (_BSC // _TCR + i, 0)),
            pl.BlockSpec((5 * _NSUB, _D), lambda i: (0, 0)),
        ],
        out_specs=pl.BlockSpec((_TCR, _D), lambda i: (i, 0)),
        out_shape=jax.ShapeDtypeStruct((_B - _BSC, _D), jnp.float32),
    )(x, tab)


def kernel(x, unnormalized_widths, unnormalized_heights,
           unnormalized_derivatives, unnormalized_lambdas):
    tab = _build_tables(unnormalized_widths, unnormalized_heights,
                        unnormalized_derivatives, unnormalized_lambdas)
    if _BSC > 0:
        tab_sc = (tab.reshape(5 * _NSUB, _NW, _CPT)
                     .transpose(1, 0, 2)
                     .reshape(_NW, 5 * _NSUB * _CPT))
        y_sc = _make_sc_spline()(x, tab_sc)
        if _BSC == _B:
            return y_sc
    y_tc = _tc_spline(x, tab)
    if _BSC == 0:
        return y_tc
    return lax.dynamic_update_slice(y_sc, y_tc, (_BSC, 0))


# speedup vs baseline: 1.0355x; 1.0355x over previous
import functools

import jax
import jax.numpy as jnp
from jax import lax
from jax.experimental import pallas as pl
from jax.experimental.pallas import tpu as pltpu
from jax.experimental.pallas import tpu_sc as plsc

_D = 4096
_B = 16384
_K = 8
_NSUB = 2 * _K
_NC = 2
_NS = 16
_NW = _NC * _NS
_CPT = _D // _NW
_R = 192

_MIN_BW = 1e-3
_MIN_D = 1e-3
_MIN_L = 0.025
_BOUND = 3.0

_BSC = 9216
_TCR = 256
_NCH = _BSC // _R


def _prep_body(uw_ref, uh_ref, ud_ref, ul_ref, tab_ref):
    w8 = jax.nn.softmax(uw_ref[:], axis=0)
    h8 = jax.nn.softmax(uh_ref[:], axis=0)
    dv = _MIN_D + jax.nn.softplus(ud_ref[:])
    lam8 = (1.0 - 2.0 * _MIN_L) * jax.nn.sigmoid(ul_ref[:]) + _MIN_L

    w8 = _MIN_BW + (1.0 - _MIN_BW * _K) * w8
    h8 = _MIN_BW + (1.0 - _MIN_BW * _K) * h8

    one = jnp.ones((1, _D), jnp.float32)

    cw = [jnp.full((1, _D), -_BOUND, jnp.float32)]
    acc = jnp.zeros((1, _D), jnp.float32)
    for k in range(_K - 1):
        acc = acc + w8[k:k + 1, :]
        cw.append(2.0 * _BOUND * acc - _BOUND)
    cw.append(jnp.full((1, _D), _BOUND, jnp.float32))

    ch = [jnp.full((1, _D), -_BOUND, jnp.float32)]
    acc = jnp.zeros((1, _D), jnp.float32)
    for k in range(_K - 1):
        acc = acc + h8[k:k + 1, :]
        ch.append(2.0 * _BOUND * acc - _BOUND)
    ch.append(jnp.full((1, _D), _BOUND, jnp.float32))

    dpad = [one] + [dv[k:k + 1, :] for k in range(_K - 1)] + [one]

    for k in range(_K):
        wk = cw[k + 1] - cw[k]
        hk = ch[k + 1] - ch[k]
        lam = lam8[k:k + 1, :]
        dk = dpad[k]
        dk1 = dpad[k + 1]
        wb = jnp.sqrt(dk / dk1)
        delta = hk / wk
        wc = (lam * dk + (1.0 - lam) * wb * dk1) / delta
        ya = ch[k]
        yb = hk + ch[k]
        yc = ((1.0 - lam) * ya + lam * wb * yb) / ((1.0 - lam) + lam * wb)

        lamw = lam * wk
        tab_ref[pl.ds(2 * k, 1), :] = cw[k]
        tab_ref[pl.ds(_NSUB + 2 * k, 1), :] = ya * lamw
        tab_ref[pl.ds(2 * _NSUB + 2 * k, 1), :] = wc * yc - ya
        tab_ref[pl.ds(3 * _NSUB + 2 * k, 1), :] = lamw
        tab_ref[pl.ds(4 * _NSUB + 2 * k, 1), :] = wc - 1.0
        wcyc = wc * yc
        wbyb = wb * yb
        om = wk - lamw
        tab_ref[pl.ds(2 * k + 1, 1), :] = cw[k] + lamw
        tab_ref[pl.ds(_NSUB + 2 * k + 1, 1), :] = wcyc * om
        tab_ref[pl.ds(2 * _NSUB + 2 * k + 1, 1), :] = wbyb - wcyc
        tab_ref[pl.ds(3 * _NSUB + 2 * k + 1, 1), :] = wc * om
        tab_ref[pl.ds(4 * _NSUB + 2 * k + 1, 1), :] = wb - wc


def _build_tables(uw, uh, ud, ul):
    return pl.pallas_call(
        _prep_body,
        out_shape=jax.ShapeDtypeStruct((5 * _NSUB, _D), jnp.float32),
    )(uw.T, uh.T, ud.T, ul.T)


def _sc_body(x_hbm, tab_hbm, out_hbm, tab_v, in0, in1, out0, out1,
             is0, is1, os0, os1):
    wid = lax.axis_index("s") * _NC + lax.axis_index("c")
    col0 = wid * _CPT

    pltpu.sync_copy(tab_hbm.at[wid], tab_v)

    def in_dma(g, buf, sem):
        pltpu.async_copy(
            x_hbm.at[pl.ds(g * _R, _R), pl.ds(col0, _CPT)], buf, sem)

    def out_dma(g, buf, sem):
        pltpu.async_copy(
            buf, out_hbm.at[pl.ds(g * _R, _R), pl.ds(col0, _CPT)], sem)

    def wait_in(buf, sem):
        pltpu.make_async_copy(
            x_hbm.at[pl.ds(0, _R), pl.ds(col0, _CPT)], buf, sem).wait()

    def wait_out(buf, sem):
        pltpu.make_async_copy(
            buf, out_hbm.at[pl.ds(0, _R), pl.ds(col0, _CPT)], sem).wait()

    in_dma(0, in0, is0)
    in_dma(1, in1, is1)

    lanes = jnp.arange(16, dtype=jnp.int32)
    half = jnp.full((16,), 8 * _CPT, jnp.int32)
    zeros = jnp.zeros((16,), jnp.int32)

    neg3 = jnp.full((16,), -_BOUND, jnp.float32)

    tab_p = tab_v.at[pl.ds(_NSUB * _CPT, _NSUB * _CPT)]
    tab_q = tab_v.at[pl.ds(2 * _NSUB * _CPT, _NSUB * _CPT)]
    tab_r = tab_v.at[pl.ds(3 * _NSUB * _CPT, _NSUB * _CPT)]
    tab_s = tab_v.at[pl.ds(4 * _NSUB * _CPT, _NSUB * _CPT)]

    def compute_chunk(ibuf, obuf):
        for g in range(_CPT // 16):
            colv = lanes + (g * 16)
            colv_hi = colv + 8 * _CPT
            b8 = tab_v[pl.ds(8 * _CPT + g * 16, 16)]

            @plsc.parallel_loop(0, _R, unroll=4)
            def row(i, g=g, colv=colv, colv_hi=colv_hi, b8=b8):
                x = ibuf[i, pl.ds(g * 16, 16)]
                m = x >= b8
                jc = jnp.where(m, colv_hi, colv)
                blo = jnp.where(m, b8, neg3)
                for step in (4 * _CPT, 2 * _CPT, _CPT):
                    probe = jc + step
                    bv = plsc.load_gather(tab_v, [probe])
                    m = x >= bv
                    jc = jnp.where(m, probe, jc)
                    blo = jnp.where(m, bv, blo)
                pv = plsc.load_gather(tab_p, [jc])
                qv = plsc.load_gather(tab_q, [jc])
                rv = plsc.load_gather(tab_r, [jc])
                sv = plsc.load_gather(tab_s, [jc])
                t = x - blo
                y = (pv + qv * t) / (rv + sv * t)
                ok = jnp.abs(x) <= _BOUND
                obuf[i, pl.ds(g * 16, 16)] = jnp.where(ok, y, x)

    def pair(pi, carry):
        for g_off, ibuf, obuf, isem, osem in (
                (0, in0, out0, is0, os0), (1, in1, out1, is1, os1)):
            g = pi * 2 + g_off
            wait_in(ibuf, isem)

            @pl.when(pi >= 1)
            def _():
                wait_out(obuf, osem)

            compute_chunk(ibuf, obuf)
            out_dma(g, obuf, osem)

            @pl.when(g + 2 < _NCH)
            def _():
                in_dma(g + 2, ibuf, isem)
        return carry

    lax.fori_loop(0, _NCH // 2, pair, 0)
    wait_out(out0, os0)
    wait_out(out1, os1)


@functools.cache
def _make_sc_spline():
    return functools.partial(
        pl.kernel,
        out_type=jax.ShapeDtypeStruct((_B, _D), jnp.float32),
        mesh=plsc.VectorSubcoreMesh(core_axis_name="c", subcore_axis_name="s"),
        compiler_params=pltpu.CompilerParams(needs_layout_passes=False),
        scratch_types=[
            pltpu.VMEM((5 * _NSUB * _CPT,), jnp.float32),
            pltpu.VMEM((_R, _CPT), jnp.float32),
            pltpu.VMEM((_R, _CPT), jnp.float32),
            pltpu.VMEM((_R, _CPT), jnp.float32),
            pltpu.VMEM((_R, _CPT), jnp.float32),
            pltpu.SemaphoreType.DMA,
            pltpu.SemaphoreType.DMA,
            pltpu.SemaphoreType.DMA,
            pltpu.SemaphoreType.DMA,
        ],
    )(_sc_body)


def _tc_spline_body(x_ref, tab_ref, o_ref):
    x = x_ref[:]
    ind = x >= tab_ref[pl.ds(1, 1), :]
    bs = jnp.where(ind, tab_ref[pl.ds(1, 1), :], tab_ref[pl.ds(0, 1), :])
    ps = jnp.where(ind, tab_ref[pl.ds(_NSUB + 1, 1), :],
                   tab_ref[pl.ds(_NSUB, 1), :])
    qs = jnp.where(ind, tab_ref[pl.ds(2 * _NSUB + 1, 1), :],
                   tab_ref[pl.ds(2 * _NSUB, 1), :])
    rs = jnp.where(ind, tab_ref[pl.ds(3 * _NSUB + 1, 1), :],
                   tab_ref[pl.ds(3 * _NSUB, 1), :])
    ss = jnp.where(ind, tab_ref[pl.ds(4 * _NSUB + 1, 1), :],
                   tab_ref[pl.ds(4 * _NSUB, 1), :])
    for m in range(2, _NSUB):
        ind = x >= tab_ref[pl.ds(m, 1), :]
        bs = jnp.where(ind, tab_ref[pl.ds(m, 1), :], bs)
        ps = jnp.where(ind, tab_ref[pl.ds(_NSUB + m, 1), :], ps)
        qs = jnp.where(ind, tab_ref[pl.ds(2 * _NSUB + m, 1), :], qs)
        rs = jnp.where(ind, tab_ref[pl.ds(3 * _NSUB + m, 1), :], rs)
        ss = jnp.where(ind, tab_ref[pl.ds(4 * _NSUB + m, 1), :], ss)
    t = x - bs
    y = (ps + qs * t) / (rs + ss * t)
    ok = jnp.abs(x) <= _BOUND
    o_ref[:] = jnp.where(ok, y, x)


def _tc_spline(x, tab):
    n_blocks = (_B - _BSC) // _TCR
    return pl.pallas_call(
        _tc_spline_body,
        grid=(n_blocks,),
        in_specs=[
            pl.BlockSpec((_TCR, _D), lambda i: (_BSC // _TCR + i, 0)),
            pl.BlockSpec((5 * _NSUB, _D), lambda i: (0, 0)),
        ],
        out_specs=pl.BlockSpec((_TCR, _D), lambda i: (i, 0)),
        out_shape=jax.ShapeDtypeStruct((_B - _BSC, _D), jnp.float32),
    )(x, tab)


def kernel(x, unnormalized_widths, unnormalized_heights,
           unnormalized_derivatives, unnormalized_lambdas):
    tab = _build_tables(unnormalized_widths, unnormalized_heights,
                        unnormalized_derivatives, unnormalized_lambdas)
    if _BSC > 0:
        tab_sc = (tab.reshape(5 * _NSUB, _NW, _CPT)
                     .transpose(1, 0, 2)
                     .reshape(_NW, 5 * _NSUB * _CPT))
        y_sc = _make_sc_spline()(x, tab_sc)
        if _BSC == _B:
            return y_sc
    y_tc = _tc_spline(x, tab)
    if _BSC == 0:
        return y_tc
    return lax.dynamic_update_slice(y_sc, y_tc, (_BSC, 0))
